# baseline (speedup 1.0000x reference)
import functools

import jax
import jax.numpy as jnp
from jax import lax
from jax.experimental import pallas as pl
from jax.experimental.pallas import tpu as pltpu
from jax.experimental.pallas import tpu_sc as plsc

B, N, T, H, W = 16, 3000, 10, 200, 200
NUM = N // 3
LANES = 16
P = 1024
NCH = P // LANES
PT = P * T
HW = H * W
BIG = 1 << 30

_mesh = plsc.VectorSubcoreMesh(core_axis_name="c", subcore_axis_name="s")


@functools.partial(
    pl.kernel,
    out_type=jax.ShapeDtypeStruct((B, 32), jnp.float32),
    mesh=_mesh,
    compiler_params=pltpu.CompilerParams(needs_layout_passes=False),
    scratch_types=[
        pltpu.VMEM((NUM * 3 * T,), jnp.float32),
        pltpu.VMEM((PT,), jnp.int32),
        pltpu.VMEM((PT,), jnp.int32),
        pltpu.VMEM((PT,), jnp.float32),
        pltpu.VMEM((PT,), jnp.float32),
        pltpu.VMEM((P,), jnp.float32),
        pltpu.VMEM((P,), jnp.float32),
        pltpu.VMEM((P,), jnp.float32),
        pltpu.VMEM((P,), jnp.float32),
        pltpu.VMEM((B,), jnp.int32),
        pltpu.VMEM((2 * B,), jnp.float32),
        pltpu.VMEM((32,), jnp.float32),
        pltpu.SemaphoreType.DMA,
        pltpu.SemaphoreType.DMA,
    ],
)
def _plan_sc(trajs_hbm, cost_hbm, sem_hbm, hd0_hbm, hd1_hbm, cmd_hbm, tp_hbm,
             out_hbm,
             traj_ref, idxt_ref, idxh_ref, va_ref, vb_ref,
             xe_ref, ye_ref, cs_ref, ss_ref,
             cmd_ref, tp_ref, row_ref,
             sem_a, sem_b):
    cidx = lax.axis_index("c")
    sidx = lax.axis_index("s")

    @pl.when(cidx == 0)
    def _body():
        b = sidx
        lane = lax.iota(jnp.int32, LANES)

        pltpu.sync_copy(cmd_hbm, cmd_ref)
        pltpu.sync_copy(tp_hbm, tp_ref)
        bvec = jnp.full((LANES,), b, jnp.int32)
        cmd = plsc.load_gather(cmd_ref, [bvec])[0]
        elem0 = (b * N + cmd * NUM) * 3 * T
        pltpu.sync_copy(trajs_hbm.at[pl.ds(elem0, NUM * 3 * T)], traj_ref)

        bofft = b * (T * HW)
        boffh = b * HW

        def t_body(t, carry):
            tvec = jnp.full((LANES,), 3 * t, jnp.int32)

            def c_body(ci, carry2):
                n = jnp.minimum(ci * LANES + lane, NUM - 1)
                x = plsc.load_gather(traj_ref, [n * (3 * T) + tvec])
                y = plsc.load_gather(traj_ref, [n * (3 * T) + tvec + 1])
                xi = jnp.clip((x * W).astype(jnp.int32), 0, W - 1)
                yi = jnp.clip((y * H).astype(jnp.int32), 0, H - 1)
                flat = yi * W + xi
                pos = t * P + ci * LANES
                idxt_ref[pl.ds(pos, LANES)] = bofft + t * HW + flat
                idxh_ref[pl.ds(pos, LANES)] = boffh + flat

                @pl.when(t == T - 1)
                def _():
                    xe_ref[pl.ds(ci * LANES, LANES)] = x
                    ye_ref[pl.ds(ci * LANES, LANES)] = y

                return carry2

            return lax.fori_loop(0, NCH, c_body, carry)

        lax.fori_loop(0, T, t_body, 0)

        cp_a = pltpu.async_copy(cost_hbm.at[idxt_ref], va_ref, sem_a)
        cp_b = pltpu.async_copy(sem_hbm.at[idxt_ref], vb_ref, sem_b)
        cp_a.wait()
        cp_b.wait()

        def acc_a(ci, carry):
            pos0 = ci * LANES

            def t_acc(t, ac):
                a, ss = ac
                p = t * P + pos0
                cv = va_ref[pl.ds(p, LANES)]
                sv = vb_ref[pl.ds(p, LANES)]
                return (a + (cv + 5.0 * sv), ss + sv)

            zero = jnp.zeros((LANES,), jnp.float32)
            a, ss = lax.fori_loop(0, T, t_acc, (zero, zero))
            cs_ref[pl.ds(pos0, LANES)] = a
            ss_ref[pl.ds(pos0, LANES)] = ss
            return carry

        lax.fori_loop(0, NCH, acc_a, 0)

        cp_a = pltpu.async_copy(hd0_hbm.at[idxh_ref], va_ref, sem_a)
        cp_b = pltpu.async_copy(hd1_hbm.at[idxh_ref], vb_ref, sem_b)
        cp_a.wait()
        cp_b.wait()

        tpx = plsc.load_gather(tp_ref, [2 * bvec])[0]
        tpy = plsc.load_gather(tp_ref, [2 * bvec + 1])[0]

        def acc_b(ci, carry):
            pos0 = ci * LANES

            def t_acc(t, a):
                p = t * P + pos0
                lv = va_ref[pl.ds(p, LANES)]
                dv = vb_ref[pl.ds(p, LANES)]
                return a + (2.0 * lv - 3.0 * dv)

            a = lax.fori_loop(0, T, t_acc, jnp.zeros((LANES,), jnp.float32))
            dx = xe_ref[pl.ds(pos0, LANES)] - tpx
            dy = ye_ref[pl.ds(pos0, LANES)] - tpy
            cs_ref[pl.ds(pos0, LANES)] = (
                cs_ref[pl.ds(pos0, LANES)] + a + dx * dx + dy * dy
            )
            return carry

        lax.fori_loop(0, NCH, acc_b, 0)

        def min_body(ci, m):
            v = cs_ref[pl.ds(ci * LANES, LANES)]
            return jnp.minimum(m, jnp.min(v))

        m = lax.fori_loop(0, NCH, min_body, jnp.float32(jnp.inf))

        def sel_body(ci, cur):
            v = cs_ref[pl.ds(ci * LANES, LANES)]
            gid = ci * LANES + lane
            cand = jnp.where(v == m, gid, jnp.int32(BIG))
            return jnp.minimum(cur, jnp.min(cand))

        sel = lax.fori_loop(0, NCH, sel_body, jnp.int32(BIG))

        def saf_body(ci, acc):
            sv = ss_ref[pl.ds(ci * LANES, LANES)]
            gid = ci * LANES + lane
            return acc + jnp.sum(jnp.where(gid == sel, sv, 0.0))

        safety = lax.fori_loop(0, NCH, saf_body, jnp.float32(0.0))

        base = jnp.full((LANES,), sel * (3 * T), jnp.int32)
        lo = plsc.load_gather(traj_ref, [base + lane])
        hi = plsc.load_gather(
            traj_ref, [base + jnp.minimum(lane + LANES, 3 * T - 1)])
        hi = jnp.where(lane == 3 * T - LANES, m, hi)
        hi = jnp.where(lane == 3 * T + 1 - LANES, safety, hi)
        row_ref[pl.ds(0, LANES)] = lo
        row_ref[pl.ds(LANES, LANES)] = hi
        pltpu.sync_copy(row_ref, out_hbm.at[b])


def kernel(cam_front, trajs, gt_trajs, cost_volume, semantic_pred, hd_map,
           commands, target_points, k):
    trajs_flat = trajs.reshape(-1)
    cost_flat = cost_volume.reshape(-1)
    sem_flat = semantic_pred.reshape(-1)
    hd0_flat = hd_map[:, 0].reshape(-1)
    hd1_flat = hd_map[:, 1].reshape(-1)
    cmds = commands.astype(jnp.int32)
    tp_flat = target_points.reshape(-1)
    return _plan_sc(trajs_flat, cost_flat, sem_flat, hd0_flat, hd1_flat,
                    cmds, tp_flat)


# speedup vs baseline: 1.0012x; 1.0012x over previous
import functools

import jax
import jax.numpy as jnp
from jax import lax
from jax.experimental import pallas as pl
from jax.experimental.pallas import tpu as pltpu
from jax.experimental.pallas import tpu_sc as plsc

B, N, T, H, W = 16, 3000, 10, 200, 200
NUM = N // 3
LANES = 16
P = 1024
NCH = P // LANES
PT = P * T
HW = H * W
BIG = 1 << 30

_mesh = plsc.VectorSubcoreMesh(core_axis_name="c", subcore_axis_name="s")


@functools.partial(
    pl.kernel,
    out_type=jax.ShapeDtypeStruct((B, 32), jnp.float32),
    mesh=_mesh,
    compiler_params=pltpu.CompilerParams(needs_layout_passes=False),
    scratch_types=[
        pltpu.VMEM((NUM * 3 * T,), jnp.float32),
        pltpu.VMEM((PT,), jnp.int32),
        pltpu.VMEM((PT,), jnp.int32),
        pltpu.VMEM((PT,), jnp.int32),
        pltpu.VMEM((PT,), jnp.float32),
        pltpu.VMEM((PT,), jnp.float32),
        pltpu.VMEM((P,), jnp.float32),
        pltpu.VMEM((P,), jnp.float32),
        pltpu.VMEM((P,), jnp.float32),
        pltpu.VMEM((P,), jnp.float32),
        pltpu.VMEM((B,), jnp.int32),
        pltpu.VMEM((2 * B,), jnp.float32),
        pltpu.VMEM((32,), jnp.float32),
        pltpu.SemaphoreType.DMA,
        pltpu.SemaphoreType.DMA,
    ],
)
def _plan_sc(trajs_hbm, cost_hbm, sem_hbm, hd_hbm, cmd_hbm, tp_hbm,
             out_hbm,
             traj_ref, idxt_ref, idxh_ref, idxd_ref, va_ref, vb_ref,
             xe_ref, ye_ref, cs_ref, ss_ref,
             cmd_ref, tp_ref, row_ref,
             sem_a, sem_b):
    cidx = lax.axis_index("c")
    sidx = lax.axis_index("s")

    @pl.when(cidx == 0)
    def _body():
        b = sidx
        lane = lax.iota(jnp.int32, LANES)

        pltpu.sync_copy(cmd_hbm, cmd_ref)
        pltpu.sync_copy(tp_hbm, tp_ref)
        bvec = jnp.full((LANES,), b, jnp.int32)
        cmd = plsc.load_gather(cmd_ref, [bvec])[0]
        elem0 = (b * N + cmd * NUM) * 3 * T
        pltpu.sync_copy(trajs_hbm.at[pl.ds(elem0, NUM * 3 * T)], traj_ref)

        bofft = b * (T * HW)
        boffh = b * (2 * HW)

        def t_body(t, carry):
            tvec = jnp.full((LANES,), 3 * t, jnp.int32)

            def c_body(ci, carry2):
                n = jnp.minimum(ci * LANES + lane, NUM - 1)
                x = plsc.load_gather(traj_ref, [n * (3 * T) + tvec])
                y = plsc.load_gather(traj_ref, [n * (3 * T) + tvec + 1])
                xi = jnp.clip((x * W).astype(jnp.int32), 0, W - 1)
                yi = jnp.clip((y * H).astype(jnp.int32), 0, H - 1)
                flat = yi * W + xi
                pos = t * P + ci * LANES
                idxt_ref[pl.ds(pos, LANES)] = bofft + t * HW + flat
                idxh_ref[pl.ds(pos, LANES)] = boffh + flat
                idxd_ref[pl.ds(pos, LANES)] = boffh + HW + flat

                @pl.when(t == T - 1)
                def _():
                    xe_ref[pl.ds(ci * LANES, LANES)] = x
                    ye_ref[pl.ds(ci * LANES, LANES)] = y

                return carry2

            return lax.fori_loop(0, NCH, c_body, carry)

        lax.fori_loop(0, T, t_body, 0)

        cp_a = pltpu.async_copy(cost_hbm.at[idxt_ref], va_ref, sem_a)
        cp_b = pltpu.async_copy(sem_hbm.at[idxt_ref], vb_ref, sem_b)
        cp_a.wait()
        cp_b.wait()

        def acc_a(ci, carry):
            pos0 = ci * LANES

            def t_acc(t, ac):
                a, ss = ac
                p = t * P + pos0
                cv = va_ref[pl.ds(p, LANES)]
                sv = vb_ref[pl.ds(p, LANES)]
                return (a + (cv + 5.0 * sv), ss + sv)

            zero = jnp.zeros((LANES,), jnp.float32)
            a, ss = lax.fori_loop(0, T, t_acc, (zero, zero))
            cs_ref[pl.ds(pos0, LANES)] = a
            ss_ref[pl.ds(pos0, LANES)] = ss
            return carry

        lax.fori_loop(0, NCH, acc_a, 0)

        cp_a = pltpu.async_copy(hd_hbm.at[idxh_ref], va_ref, sem_a)
        cp_b = pltpu.async_copy(hd_hbm.at[idxd_ref], vb_ref, sem_b)
        cp_a.wait()
        cp_b.wait()

        tpx = plsc.load_gather(tp_ref, [2 * bvec])[0]
        tpy = plsc.load_gather(tp_ref, [2 * bvec + 1])[0]

        def acc_b(ci, carry):
            pos0 = ci * LANES

            def t_acc(t, a):
                p = t * P + pos0
                lv = va_ref[pl.ds(p, LANES)]
                dv = vb_ref[pl.ds(p, LANES)]
                return a + (2.0 * lv - 3.0 * dv)

            a = lax.fori_loop(0, T, t_acc, jnp.zeros((LANES,), jnp.float32))
            dx = xe_ref[pl.ds(pos0, LANES)] - tpx
            dy = ye_ref[pl.ds(pos0, LANES)] - tpy
            cs_ref[pl.ds(pos0, LANES)] = (
                cs_ref[pl.ds(pos0, LANES)] + a + dx * dx + dy * dy
            )
            return carry

        lax.fori_loop(0, NCH, acc_b, 0)

        def min_body(ci, m):
            v = cs_ref[pl.ds(ci * LANES, LANES)]
            return jnp.minimum(m, jnp.min(v))

        m = lax.fori_loop(0, NCH, min_body, jnp.float32(jnp.inf))

        def sel_body(ci, cur):
            v = cs_ref[pl.ds(ci * LANES, LANES)]
            gid = ci * LANES + lane
            cand = jnp.where(v == m, gid, jnp.int32(BIG))
            return jnp.minimum(cur, jnp.min(cand))

        sel = lax.fori_loop(0, NCH, sel_body, jnp.int32(BIG))

        def saf_body(ci, acc):
            sv = ss_ref[pl.ds(ci * LANES, LANES)]
            gid = ci * LANES + lane
            return acc + jnp.sum(jnp.where(gid == sel, sv, 0.0))

        safety = lax.fori_loop(0, NCH, saf_body, jnp.float32(0.0))

        base = jnp.full((LANES,), sel * (3 * T), jnp.int32)
        lo = plsc.load_gather(traj_ref, [base + lane])
        hi = plsc.load_gather(
            traj_ref, [base + jnp.minimum(lane + LANES, 3 * T - 1)])
        hi = jnp.where(lane == 3 * T - LANES, m, hi)
        hi = jnp.where(lane == 3 * T + 1 - LANES, safety, hi)
        row_ref[pl.ds(0, LANES)] = lo
        row_ref[pl.ds(LANES, LANES)] = hi
        pltpu.sync_copy(row_ref, out_hbm.at[b])


def kernel(cam_front, trajs, gt_trajs, cost_volume, semantic_pred, hd_map,
           commands, target_points, k):
    trajs_flat = trajs.reshape(-1)
    cost_flat = cost_volume.reshape(-1)
    sem_flat = semantic_pred.reshape(-1)
    hd_flat = hd_map.reshape(-1)
    cmds = commands.astype(jnp.int32)
    tp_flat = target_points.reshape(-1)
    return _plan_sc(trajs_flat, cost_flat, sem_flat, hd_flat,
                    cmds, tp_flat)
